# baseline (speedup 1.0000x reference)
import functools

import jax
import jax.numpy as jnp
from jax import lax
from jax.experimental import pallas as pl
from jax.experimental.pallas import tpu as pltpu
from jax.experimental.pallas import tpu_sc as plsc

N_TOKENS = 4096
D_MODEL = 1024
NUM_EXPERTS = 8
TB = 512
B = 256
M = 2 * N_TOKENS
NBLK = M // B + NUM_EXPERTS
M_PAD = NBLK * B
AROWS = M // 128


def _router_body(x_ref, rw_ref, rb_ref, i1_ref, i2_ref, w1_ref, w2_ref):
    logits = jax.lax.dot_general(
        x_ref[...].astype(jnp.bfloat16), rw_ref[...].astype(jnp.bfloat16),
        (((1,), (1,)), ((), ())),
        preferred_element_type=jnp.float32,
    ) + rb_ref[...]
    m = jnp.max(logits, axis=-1, keepdims=True)
    eu = jnp.exp(logits - m)
    p = eu / jnp.sum(eu, axis=-1, keepdims=True)
    iota = jax.lax.broadcasted_iota(jnp.int32, p.shape, 1)
    p1 = jnp.max(p, axis=-1, keepdims=True)
    i1 = jnp.argmax(p, axis=-1)[:, None].astype(jnp.int32)
    pm = jnp.where(iota == i1, -1.0, p)
    p2 = jnp.max(pm, axis=-1, keepdims=True)
    i2 = jnp.argmax(pm, axis=-1)[:, None].astype(jnp.int32)
    s = p1 + p2
    i1_ref[...] = i1
    i2_ref[...] = i2
    w1_ref[...] = p1 / s
    w2_ref[...] = p2 / s


def _dispatch_body(e2d_ref, pos_ref, blk_ref, tok_ref):
    e2d = e2d_ref[...]
    pos = jnp.zeros((AROWS, 128), jnp.int32)
    blk_acc = jnp.zeros((1, 128), jnp.int32)
    lane = jax.lax.broadcasted_iota(jnp.int32, (1, 128), 1)
    start_rows = jnp.int32(0)
    start_blk = jnp.int32(0)
    for e in range(NUM_EXPERTS):
        msk = (e2d == e).astype(jnp.int32)
        c = msk
        for s in (1, 2, 4, 8, 16, 32, 64):
            c = c + jnp.concatenate(
                [jnp.zeros((AROWS, s), jnp.int32), c[:, :128 - s]], axis=1)
        tot = c[:, 127:128]
        t = tot
        for s in (1, 2, 4, 8, 16, 32):
            t = t + jnp.concatenate(
                [jnp.zeros((s, 1), jnp.int32), t[:AROWS - s, :]], axis=0)
        rank_incl = c + (t - tot)
        cnt = jnp.sum(msk)
        blk_acc = blk_acc + (lane >= start_blk).astype(jnp.int32)
        pos = pos + msk * (rank_incl - 1 + start_rows)
        nblk_e = (cnt + (B - 1)) // B
        start_blk = start_blk + nblk_e
        start_rows = start_rows + nblk_e * B
    pos_ref[...] = pos
    blk_ref[...] = jnp.minimum(blk_acc - 1, NUM_EXPERTS - 1)
    r_iota = jax.lax.broadcasted_iota(jnp.int32, (AROWS, 128), 0)
    c_iota = jax.lax.broadcasted_iota(jnp.int32, (AROWS, 128), 1)
    tok_ref[...] = (r_iota % (N_TOKENS // 128)) * 128 + c_iota


def _make_scatter_kernel():
    mesh = plsc.VectorSubcoreMesh(core_axis_name="c", subcore_axis_name="s")

    @functools.partial(
        pl.kernel, mesh=mesh,
        out_type=[
            jax.ShapeDtypeStruct((M_PAD, 4, 128), jnp.int32),
            jax.ShapeDtypeStruct((M_PAD,), jnp.float32),
        ],
        scratch_types=[
            pltpu.VMEM((128,), jnp.int32),
            pltpu.VMEM((128,), jnp.int32),
            pltpu.VMEM((128,), jnp.float32),
            pltpu.VMEM((128, 4, 128), jnp.int32),
            pltpu.SemaphoreType.DMA,
        ],
    )
    def scatter_kernel(x_hbm, pos_hbm, tok_hbm, w_hbm, xs_hbm, ws_hbm,
                       idx_v, tok_v, w_v, rows_v, sem):
        wid = lax.axis_index("s") * 2 + lax.axis_index("c")
        for j in range(2):
            r = wid * 2 + j
            pltpu.sync_copy(pos_hbm.at[r], idx_v)
            pltpu.sync_copy(tok_hbm.at[r], tok_v)
            pltpu.sync_copy(w_hbm.at[r], w_v)
            pltpu.async_copy(x_hbm.at[tok_v], rows_v, sem).wait()
            pltpu.async_copy(rows_v, xs_hbm.at[idx_v], sem).wait()
            pltpu.async_copy(w_v, ws_hbm.at[idx_v], sem).wait()

    return scatter_kernel


def _gmm_body(be_smem, xs_ref, w_ref, b_ref, ws_ref, y_ref):
    pre = jax.lax.dot_general(
        xs_ref[...], w_ref[0], (((1,), (1,)), ((), ())),
        preferred_element_type=jnp.float32,
    ) + b_ref[0, 0][None, :]
    act = 0.5 * pre * (1.0 + jax.lax.erf(pre * 0.7071067811865476))
    y_ref[...] = (act * ws_ref[...]).astype(jnp.bfloat16)


def _make_combine_kernel():
    mesh = plsc.VectorSubcoreMesh(core_axis_name="c", subcore_axis_name="s")

    @functools.partial(
        pl.kernel, mesh=mesh,
        out_type=[
            jax.ShapeDtypeStruct((N_TOKENS, 4, 128), jnp.int32),
            jax.ShapeDtypeStruct((N_TOKENS, 4, 128), jnp.int32),
        ],
        scratch_types=[
            pltpu.VMEM((128,), jnp.int32),
            pltpu.VMEM((128, 4, 128), jnp.int32),
            pltpu.SemaphoreType.DMA,
        ],
    )
    def combine_kernel(y_hbm, pos_hbm, y1_hbm, y2_hbm, idx_v, rows_v, sem):
        wid = lax.axis_index("s") * 2 + lax.axis_index("c")
        base = wid * 128
        pltpu.sync_copy(pos_hbm.at[wid], idx_v)
        pltpu.async_copy(y_hbm.at[idx_v], rows_v, sem).wait()
        pltpu.sync_copy(rows_v, y1_hbm.at[pl.ds(base, 128)])
        pltpu.sync_copy(pos_hbm.at[(N_TOKENS // 128) + wid], idx_v)
        pltpu.async_copy(y_hbm.at[idx_v], rows_v, sem).wait()
        pltpu.sync_copy(rows_v, y2_hbm.at[pl.ds(base, 128)])

    return combine_kernel


def _add_body(y1_ref, y2_ref, out_ref):
    out_ref[...] = (y1_ref[...].astype(jnp.float32)
                    + y2_ref[...].astype(jnp.float32))


@jax.jit
def kernel(x, router_w, router_b, expert_w, expert_b):
    i1, i2, w1, w2 = pl.pallas_call(
        _router_body,
        grid=(N_TOKENS // TB,),
        in_specs=[
            pl.BlockSpec((TB, D_MODEL), lambda t: (t, 0)),
            pl.BlockSpec((NUM_EXPERTS, D_MODEL), lambda t: (0, 0)),
            pl.BlockSpec((1, NUM_EXPERTS), lambda t: (0, 0)),
        ],
        out_specs=[
            pl.BlockSpec((TB, 1), lambda t: (t, 0)),
            pl.BlockSpec((TB, 1), lambda t: (t, 0)),
            pl.BlockSpec((TB, 1), lambda t: (t, 0)),
            pl.BlockSpec((TB, 1), lambda t: (t, 0)),
        ],
        out_shape=[
            jax.ShapeDtypeStruct((N_TOKENS, 1), jnp.int32),
            jax.ShapeDtypeStruct((N_TOKENS, 1), jnp.int32),
            jax.ShapeDtypeStruct((N_TOKENS, 1), jnp.float32),
            jax.ShapeDtypeStruct((N_TOKENS, 1), jnp.float32),
        ],
    )(x, router_w, router_b.reshape(1, NUM_EXPERTS))

    e2d = jnp.concatenate(
        [i1.reshape(N_TOKENS // 128, 128), i2.reshape(N_TOKENS // 128, 128)],
        axis=0)
    w2d = jnp.concatenate(
        [w1.reshape(N_TOKENS // 128, 128), w2.reshape(N_TOKENS // 128, 128)],
        axis=0)

    pos2d, blk2d, tok2d = pl.pallas_call(
        _dispatch_body,
        out_shape=[
            jax.ShapeDtypeStruct((AROWS, 128), jnp.int32),
            jax.ShapeDtypeStruct((1, 128), jnp.int32),
            jax.ShapeDtypeStruct((AROWS, 128), jnp.int32),
        ],
    )(e2d)
    blk_expert = blk2d.reshape(128)[:NBLK]

    x_i32 = lax.bitcast_convert_type(
        x.astype(jnp.bfloat16).reshape(N_TOKENS, 512, 2), jnp.int32
    ).reshape(N_TOKENS, 4, 128)
    xs, ws = _make_scatter_kernel()(x_i32, pos2d, tok2d, w2d)
    xs_bf = lax.bitcast_convert_type(
        xs.reshape(M_PAD, 512), jnp.bfloat16).reshape(M_PAD, D_MODEL)

    y = pl.pallas_call(
        _gmm_body,
        grid_spec=pltpu.PrefetchScalarGridSpec(
            num_scalar_prefetch=1,
            grid=(NBLK,),
            in_specs=[
                pl.BlockSpec((B, D_MODEL), lambda b, be: (b, 0)),
                pl.BlockSpec((1, D_MODEL, D_MODEL), lambda b, be: (be[b], 0, 0)),
                pl.BlockSpec((1, 1, D_MODEL), lambda b, be: (be[b], 0, 0)),
                pl.BlockSpec((B, 1), lambda b, be: (b, 0)),
            ],
            out_specs=pl.BlockSpec((B, D_MODEL), lambda b, be: (b, 0)),
        ),
        out_shape=jax.ShapeDtypeStruct((M_PAD, D_MODEL), jnp.bfloat16),
        compiler_params=pltpu.CompilerParams(
            dimension_semantics=("arbitrary",),
        ),
    )(blk_expert, xs_bf,
      expert_w.astype(jnp.bfloat16),
      expert_b.reshape(NUM_EXPERTS, 1, D_MODEL),
      ws.reshape(M_PAD, 1))

    y_i32 = lax.bitcast_convert_type(
        y.reshape(M_PAD, 512, 2), jnp.int32).reshape(M_PAD, 4, 128)
    y1, y2 = _make_combine_kernel()(y_i32, pos2d)
    y1 = lax.bitcast_convert_type(
        y1.reshape(N_TOKENS, 512), jnp.bfloat16).reshape(N_TOKENS, D_MODEL)
    y2 = lax.bitcast_convert_type(
        y2.reshape(N_TOKENS, 512), jnp.bfloat16).reshape(N_TOKENS, D_MODEL)

    return pl.pallas_call(
        _add_body,
        grid=(4,),
        in_specs=[
            pl.BlockSpec((N_TOKENS // 4, D_MODEL), lambda t: (t, 0)),
            pl.BlockSpec((N_TOKENS // 4, D_MODEL), lambda t: (t, 0)),
        ],
        out_specs=pl.BlockSpec((N_TOKENS // 4, D_MODEL), lambda t: (t, 0)),
        out_shape=jax.ShapeDtypeStruct((N_TOKENS, D_MODEL), jnp.float32),
    )(y1, y2)


# speedup vs baseline: 4.0221x; 4.0221x over previous
import functools

import jax
import jax.numpy as jnp
from jax import lax
from jax.experimental import pallas as pl
from jax.experimental.pallas import tpu as pltpu
from jax.experimental.pallas import tpu_sc as plsc

N_TOKENS = 4096
D_MODEL = 1024
NUM_EXPERTS = 8
TB = 512
B = 256
M = 2 * N_TOKENS
NBLK = M // B + NUM_EXPERTS
M_PAD = NBLK * B
AROWS = M // 128
TROWS = N_TOKENS // 128
CH = 32


def _router_body(x_ref, rw_ref, rb_ref, i1_ref, i2_ref, w1_ref, w2_ref):
    logits = jax.lax.dot_general(
        x_ref[...].astype(jnp.bfloat16), rw_ref[...].astype(jnp.bfloat16),
        (((1,), (1,)), ((), ())),
        preferred_element_type=jnp.float32,
    ) + rb_ref[...]
    m = jnp.max(logits, axis=-1, keepdims=True)
    eu = jnp.exp(logits - m)
    p = eu / jnp.sum(eu, axis=-1, keepdims=True)
    iota = jax.lax.broadcasted_iota(jnp.int32, p.shape, 1)
    p1 = jnp.max(p, axis=-1, keepdims=True)
    i1 = jnp.argmax(p, axis=-1)[:, None].astype(jnp.int32)
    pm = jnp.where(iota == i1, -1.0, p)
    p2 = jnp.max(pm, axis=-1, keepdims=True)
    i2 = jnp.argmax(pm, axis=-1)[:, None].astype(jnp.int32)
    s = p1 + p2
    i1_ref[...] = i1
    i2_ref[...] = i2
    w1_ref[...] = p1 / s
    w2_ref[...] = p2 / s


def _dispatch_body(e2d_ref, pos_ref, blk_ref):
    e2d = e2d_ref[...]
    pos = jnp.zeros((AROWS, 128), jnp.int32)
    blk_acc = jnp.zeros((1, 128), jnp.int32)
    lane = jax.lax.broadcasted_iota(jnp.int32, (1, 128), 1)
    start_rows = jnp.int32(0)
    start_blk = jnp.int32(0)
    for e in range(NUM_EXPERTS):
        msk = (e2d == e).astype(jnp.int32)
        c = msk
        for s in (1, 2, 4, 8, 16, 32, 64):
            c = c + jnp.concatenate(
                [jnp.zeros((AROWS, s), jnp.int32), c[:, :128 - s]], axis=1)
        tot = c[:, 127:128]
        t = tot
        for s in (1, 2, 4, 8, 16, 32):
            t = t + jnp.concatenate(
                [jnp.zeros((s, 1), jnp.int32), t[:AROWS - s, :]], axis=0)
        rank_incl = c + (t - tot)
        cnt = jnp.sum(msk)
        blk_acc = blk_acc + (lane >= start_blk).astype(jnp.int32)
        pos = pos + msk * (rank_incl - 1 + start_rows)
        nblk_e = (cnt + (B - 1)) // B
        start_blk = start_blk + nblk_e
        start_rows = start_rows + nblk_e * B
    pos_ref[...] = pos
    blk_ref[...] = jnp.minimum(blk_acc - 1, NUM_EXPERTS - 1)


def _make_scatter_kernel():
    mesh = plsc.VectorSubcoreMesh(core_axis_name="c", subcore_axis_name="s")

    @functools.partial(
        pl.kernel, mesh=mesh,
        out_type=[
            jax.ShapeDtypeStruct((M_PAD, D_MODEL), jnp.float32),
            jax.ShapeDtypeStruct((M_PAD,), jnp.float32),
        ],
        scratch_types=[
            pltpu.VMEM((CH,), jnp.int32),
            pltpu.VMEM((CH,), jnp.int32),
            pltpu.VMEM((CH,), jnp.float32),
            pltpu.VMEM((CH,), jnp.float32),
            pltpu.VMEM((CH, D_MODEL), jnp.float32),
            pltpu.VMEM((CH, D_MODEL), jnp.float32),
            pltpu.SemaphoreType.DMA,
            pltpu.SemaphoreType.DMA,
            pltpu.SemaphoreType.DMA,
        ],
    )
    def scatter_kernel(x_hbm, pos_hbm, w_hbm, xs_hbm, ws_hbm,
                       idx_a, idx_b, w_a, w_b, rows_a, rows_b,
                       sem_a, sem_b, sem_w):
        wid = lax.axis_index("s") * 2 + lax.axis_index("c")
        idx_v = [idx_a, idx_b]
        w_v = [w_a, w_b]
        rows_v = [rows_a, rows_b]
        sems = [sem_a, sem_b]
        loads = [None, None]
        for q in range(8):
            r = wid * 2 + q // 4
            h = (q % 4) * CH
            tok0 = (r % TROWS) * 128 + h
            buf = q % 2
            pltpu.sync_copy(pos_hbm.at[r, pl.ds(h, CH)], idx_v[buf])
            pltpu.sync_copy(w_hbm.at[r, pl.ds(h, CH)], w_v[buf])
            loads[buf] = pltpu.async_copy(
                x_hbm.at[pl.ds(tok0, CH)], rows_v[buf], sems[buf])
            if q % 2 == 1:
                for b in (0, 1):
                    loads[b].wait()
                    pltpu.async_copy(
                        rows_v[b], xs_hbm.at[idx_v[b]], sems[b]).wait()
                    pltpu.async_copy(
                        w_v[b], ws_hbm.at[idx_v[b]], sem_w).wait()

    return scatter_kernel


def _gmm_body(be_smem, xs_ref, w_ref, b_ref, ws_ref, y_ref):
    pre = jax.lax.dot_general(
        xs_ref[...].astype(jnp.bfloat16), w_ref[0].astype(jnp.bfloat16),
        (((1,), (1,)), ((), ())),
        preferred_element_type=jnp.float32,
    ) + b_ref[0, 0][None, :]
    act = 0.5 * pre * (1.0 + jax.lax.erf(pre * 0.7071067811865476))
    y_ref[...] = act * ws_ref[...]


def _make_combine_kernel():
    mesh = plsc.VectorSubcoreMesh(core_axis_name="c", subcore_axis_name="s")

    @functools.partial(
        pl.kernel, mesh=mesh,
        out_type=[
            jax.ShapeDtypeStruct((N_TOKENS, D_MODEL), jnp.float32),
            jax.ShapeDtypeStruct((N_TOKENS, D_MODEL), jnp.float32),
        ],
        scratch_types=[
            pltpu.VMEM((CH,), jnp.int32),
            pltpu.VMEM((CH,), jnp.int32),
            pltpu.VMEM((CH, D_MODEL), jnp.float32),
            pltpu.VMEM((CH, D_MODEL), jnp.float32),
            pltpu.SemaphoreType.DMA,
            pltpu.SemaphoreType.DMA,
        ],
    )
    def combine_kernel(y_hbm, pos_hbm, y1_hbm, y2_hbm,
                       idx_a, idx_b, rows_a, rows_b, sem_a, sem_b):
        wid = lax.axis_index("s") * 2 + lax.axis_index("c")
        idx_v = [idx_a, idx_b]
        rows_v = [rows_a, rows_b]
        sems = [sem_a, sem_b]
        outs = [y1_hbm, y2_hbm]
        for k in range(2):
            for h in range(4):
                buf = h % 2
                base = wid * 128 + h * CH
                pltpu.sync_copy(
                    pos_hbm.at[k * TROWS + wid, pl.ds(h * CH, CH)], idx_v[buf])
                pltpu.async_copy(
                    y_hbm.at[idx_v[buf]], rows_v[buf], sems[buf]).wait()
                pltpu.sync_copy(rows_v[buf], outs[k].at[pl.ds(base, CH)])

    return combine_kernel


def _add_body(y1_ref, y2_ref, out_ref):
    out_ref[...] = y1_ref[...] + y2_ref[...]


@jax.jit
def kernel(x, router_w, router_b, expert_w, expert_b):
    i1, i2, w1, w2 = pl.pallas_call(
        _router_body,
        grid=(N_TOKENS // TB,),
        in_specs=[
            pl.BlockSpec((TB, D_MODEL), lambda t: (t, 0)),
            pl.BlockSpec((NUM_EXPERTS, D_MODEL), lambda t: (0, 0)),
            pl.BlockSpec((1, NUM_EXPERTS), lambda t: (0, 0)),
        ],
        out_specs=[
            pl.BlockSpec((TB, 1), lambda t: (t, 0)),
            pl.BlockSpec((TB, 1), lambda t: (t, 0)),
            pl.BlockSpec((TB, 1), lambda t: (t, 0)),
            pl.BlockSpec((TB, 1), lambda t: (t, 0)),
        ],
        out_shape=[
            jax.ShapeDtypeStruct((N_TOKENS, 1), jnp.int32),
            jax.ShapeDtypeStruct((N_TOKENS, 1), jnp.int32),
            jax.ShapeDtypeStruct((N_TOKENS, 1), jnp.float32),
            jax.ShapeDtypeStruct((N_TOKENS, 1), jnp.float32),
        ],
    )(x, router_w, router_b.reshape(1, NUM_EXPERTS))

    e2d = jnp.concatenate(
        [i1.reshape(TROWS, 128), i2.reshape(TROWS, 128)], axis=0)
    w2d = jnp.concatenate(
        [w1.reshape(TROWS, 128), w2.reshape(TROWS, 128)], axis=0)

    pos2d, blk2d = pl.pallas_call(
        _dispatch_body,
        out_shape=[
            jax.ShapeDtypeStruct((AROWS, 128), jnp.int32),
            jax.ShapeDtypeStruct((1, 128), jnp.int32),
        ],
    )(e2d)
    blk_expert = blk2d.reshape(128)[:NBLK]

    xs, ws = _make_scatter_kernel()(x, pos2d, w2d)

    y = pl.pallas_call(
        _gmm_body,
        grid_spec=pltpu.PrefetchScalarGridSpec(
            num_scalar_prefetch=1,
            grid=(NBLK,),
            in_specs=[
                pl.BlockSpec((B, D_MODEL), lambda b, be: (b, 0)),
                pl.BlockSpec((1, D_MODEL, D_MODEL), lambda b, be: (be[b], 0, 0)),
                pl.BlockSpec((1, 1, D_MODEL), lambda b, be: (be[b], 0, 0)),
                pl.BlockSpec((B, 1), lambda b, be: (b, 0)),
            ],
            out_specs=pl.BlockSpec((B, D_MODEL), lambda b, be: (b, 0)),
        ),
        out_shape=jax.ShapeDtypeStruct((M_PAD, D_MODEL), jnp.float32),
        compiler_params=pltpu.CompilerParams(
            dimension_semantics=("arbitrary",),
        ),
    )(blk_expert, xs, expert_w,
      expert_b.reshape(NUM_EXPERTS, 1, D_MODEL),
      ws.reshape(M_PAD, 1))

    y1, y2 = _make_combine_kernel()(y, pos2d)

    return pl.pallas_call(
        _add_body,
        grid=(4,),
        in_specs=[
            pl.BlockSpec((N_TOKENS // 4, D_MODEL), lambda t: (t, 0)),
            pl.BlockSpec((N_TOKENS // 4, D_MODEL), lambda t: (t, 0)),
        ],
        out_specs=pl.BlockSpec((N_TOKENS // 4, D_MODEL), lambda t: (t, 0)),
        out_shape=jax.ShapeDtypeStruct((N_TOKENS, D_MODEL), jnp.float32),
    )(y1, y2)


# speedup vs baseline: 4.7285x; 1.1756x over previous
import functools

import jax
import jax.numpy as jnp
from jax import lax
from jax.experimental import pallas as pl
from jax.experimental.pallas import tpu as pltpu
from jax.experimental.pallas import tpu_sc as plsc

N_TOKENS = 4096
D_MODEL = 1024
NUM_EXPERTS = 8
TB = 512
B = 256
M = 2 * N_TOKENS
NBLK = M // B + NUM_EXPERTS
M_PAD = NBLK * B
AROWS = M // 128
TROWS = N_TOKENS // 128
CH = 64
DH = D_MODEL // 2
HMASK = -65536


def _pack(lo_f32, hi_f32):
    lo_i = lax.bitcast_convert_type(
        lo_f32.astype(jnp.bfloat16).astype(jnp.float32), jnp.int32)
    hi_i = lax.bitcast_convert_type(
        hi_f32.astype(jnp.bfloat16).astype(jnp.float32), jnp.int32)
    return lax.shift_right_logical(lo_i, 16) | (hi_i & HMASK)


def _unpack(pk):
    lo = lax.bitcast_convert_type(lax.shift_left(pk, 16), jnp.float32)
    hi = lax.bitcast_convert_type(pk & HMASK, jnp.float32)
    return lo, hi


def _router_body(x_ref, rw_ref, rb_ref, i1_ref, i2_ref, w1_ref, w2_ref,
                 xi_ref):
    logits = jax.lax.dot_general(
        x_ref[...].astype(jnp.bfloat16), rw_ref[...].astype(jnp.bfloat16),
        (((1,), (1,)), ((), ())),
        preferred_element_type=jnp.float32,
    ) + rb_ref[...]
    m = jnp.max(logits, axis=-1, keepdims=True)
    eu = jnp.exp(logits - m)
    p = eu / jnp.sum(eu, axis=-1, keepdims=True)
    iota = jax.lax.broadcasted_iota(jnp.int32, p.shape, 1)
    p1 = jnp.max(p, axis=-1, keepdims=True)
    i1 = jnp.argmax(p, axis=-1)[:, None].astype(jnp.int32)
    pm = jnp.where(iota == i1, -1.0, p)
    p2 = jnp.max(pm, axis=-1, keepdims=True)
    i2 = jnp.argmax(pm, axis=-1)[:, None].astype(jnp.int32)
    s = p1 + p2
    i1_ref[...] = i1
    i2_ref[...] = i2
    w1_ref[...] = p1 / s
    w2_ref[...] = p2 / s
    xi_ref[...] = _pack(x_ref[:, :DH], x_ref[:, DH:])


def _dispatch_body(e2d_ref, pos_ref, blk_ref):
    e2d = e2d_ref[...]
    pos = jnp.zeros((AROWS, 128), jnp.int32)
    blk_acc = jnp.zeros((1, 128), jnp.int32)
    lane = jax.lax.broadcasted_iota(jnp.int32, (1, 128), 1)
    start_rows = jnp.int32(0)
    start_blk = jnp.int32(0)
    for e in range(NUM_EXPERTS):
        msk = (e2d == e).astype(jnp.int32)
        c = msk
        for s in (1, 2, 4, 8, 16, 32, 64):
            c = c + jnp.concatenate(
                [jnp.zeros((AROWS, s), jnp.int32), c[:, :128 - s]], axis=1)
        tot = c[:, 127:128]
        t = tot
        for s in (1, 2, 4, 8, 16, 32):
            t = t + jnp.concatenate(
                [jnp.zeros((s, 1), jnp.int32), t[:AROWS - s, :]], axis=0)
        rank_incl = c + (t - tot)
        cnt = jnp.sum(msk)
        blk_acc = blk_acc + (lane >= start_blk).astype(jnp.int32)
        pos = pos + msk * (rank_incl - 1 + start_rows)
        nblk_e = (cnt + (B - 1)) // B
        start_blk = start_blk + nblk_e
        start_rows = start_rows + nblk_e * B
    pos_ref[...] = pos
    blk_ref[...] = jnp.minimum(blk_acc - 1, NUM_EXPERTS - 1)


def _make_scatter_kernel():
    mesh = plsc.VectorSubcoreMesh(core_axis_name="c", subcore_axis_name="s")

    @functools.partial(
        pl.kernel, mesh=mesh,
        out_type=[
            jax.ShapeDtypeStruct((M_PAD, DH), jnp.int32),
            jax.ShapeDtypeStruct((M_PAD,), jnp.float32),
        ],
        scratch_types=[
            pltpu.VMEM((CH,), jnp.int32),
            pltpu.VMEM((CH,), jnp.int32),
            pltpu.VMEM((CH,), jnp.float32),
            pltpu.VMEM((CH,), jnp.float32),
            pltpu.VMEM((CH, DH), jnp.int32),
            pltpu.VMEM((CH, DH), jnp.int32),
            pltpu.SemaphoreType.DMA,
            pltpu.SemaphoreType.DMA,
            pltpu.SemaphoreType.DMA,
        ],
    )
    def scatter_kernel(x_hbm, pos_hbm, w_hbm, xs_hbm, ws_hbm,
                       idx_a, idx_b, w_a, w_b, rows_a, rows_b,
                       sem_a, sem_b, sem_w):
        wid = lax.axis_index("s") * 2 + lax.axis_index("c")
        idx_v = [idx_a, idx_b]
        w_v = [w_a, w_b]
        rows_v = [rows_a, rows_b]
        sems = [sem_a, sem_b]
        loads = [None, None]
        for q in range(4):
            r = wid * 2 + q // 2
            h = (q % 2) * CH
            tok0 = (r % TROWS) * 128 + h
            buf = q % 2
            pltpu.sync_copy(pos_hbm.at[r, pl.ds(h, CH)], idx_v[buf])
            pltpu.sync_copy(w_hbm.at[r, pl.ds(h, CH)], w_v[buf])
            loads[buf] = pltpu.async_copy(
                x_hbm.at[pl.ds(tok0, CH)], rows_v[buf], sems[buf])
            if q % 2 == 1:
                for b in (0, 1):
                    loads[b].wait()
                    pltpu.async_copy(
                        rows_v[b], xs_hbm.at[idx_v[b]], sems[b]).wait()
                    pltpu.async_copy(
                        w_v[b], ws_hbm.at[idx_v[b]], sem_w).wait()

    return scatter_kernel


def _gmm_body(be_smem, xs_ref, w_ref, b_ref, ws_ref, y_ref):
    a, bb = _unpack(xs_ref[...])
    w = w_ref[0]
    pre = jax.lax.dot_general(
        a.astype(jnp.bfloat16), w[:, :DH].astype(jnp.bfloat16),
        (((1,), (1,)), ((), ())),
        preferred_element_type=jnp.float32,
    ) + jax.lax.dot_general(
        bb.astype(jnp.bfloat16), w[:, DH:].astype(jnp.bfloat16),
        (((1,), (1,)), ((), ())),
        preferred_element_type=jnp.float32,
    ) + b_ref[0, 0][None, :]
    act = 0.5 * pre * (1.0 + jax.lax.erf(pre * 0.7071067811865476))
    y = act * ws_ref[...]
    y_ref[...] = _pack(y[:, :DH], y[:, DH:])


def _make_combine_kernel():
    mesh = plsc.VectorSubcoreMesh(core_axis_name="c", subcore_axis_name="s")

    @functools.partial(
        pl.kernel, mesh=mesh,
        out_type=[
            jax.ShapeDtypeStruct((N_TOKENS, DH), jnp.int32),
            jax.ShapeDtypeStruct((N_TOKENS, DH), jnp.int32),
        ],
        scratch_types=[
            pltpu.VMEM((CH,), jnp.int32),
            pltpu.VMEM((CH,), jnp.int32),
            pltpu.VMEM((CH, DH), jnp.int32),
            pltpu.VMEM((CH, DH), jnp.int32),
            pltpu.SemaphoreType.DMA,
            pltpu.SemaphoreType.DMA,
        ],
    )
    def combine_kernel(y_hbm, pos_hbm, y1_hbm, y2_hbm,
                       idx_a, idx_b, rows_a, rows_b, sem_a, sem_b):
        wid = lax.axis_index("s") * 2 + lax.axis_index("c")
        idx_v = [idx_a, idx_b]
        rows_v = [rows_a, rows_b]
        sems = [sem_a, sem_b]
        outs = [y1_hbm, y2_hbm]
        for k in range(2):
            for h in range(2):
                buf = h
                base = wid * 128 + h * CH
                pltpu.sync_copy(
                    pos_hbm.at[k * TROWS + wid, pl.ds(h * CH, CH)], idx_v[buf])
                pltpu.async_copy(
                    y_hbm.at[idx_v[buf]], rows_v[buf], sems[buf]).wait()
                pltpu.sync_copy(rows_v[buf], outs[k].at[pl.ds(base, CH)])

    return combine_kernel


def _add_body(y1_ref, y2_ref, out_ref):
    lo1, hi1 = _unpack(y1_ref[...])
    lo2, hi2 = _unpack(y2_ref[...])
    out_ref[:, :DH] = lo1 + lo2
    out_ref[:, DH:] = hi1 + hi2


@jax.jit
def kernel(x, router_w, router_b, expert_w, expert_b):
    i1, i2, w1, w2, xi = pl.pallas_call(
        _router_body,
        grid=(N_TOKENS // TB,),
        in_specs=[
            pl.BlockSpec((TB, D_MODEL), lambda t: (t, 0)),
            pl.BlockSpec((NUM_EXPERTS, D_MODEL), lambda t: (0, 0)),
            pl.BlockSpec((1, NUM_EXPERTS), lambda t: (0, 0)),
        ],
        out_specs=[
            pl.BlockSpec((TB, 1), lambda t: (t, 0)),
            pl.BlockSpec((TB, 1), lambda t: (t, 0)),
            pl.BlockSpec((TB, 1), lambda t: (t, 0)),
            pl.BlockSpec((TB, 1), lambda t: (t, 0)),
            pl.BlockSpec((TB, DH), lambda t: (t, 0)),
        ],
        out_shape=[
            jax.ShapeDtypeStruct((N_TOKENS, 1), jnp.int32),
            jax.ShapeDtypeStruct((N_TOKENS, 1), jnp.int32),
            jax.ShapeDtypeStruct((N_TOKENS, 1), jnp.float32),
            jax.ShapeDtypeStruct((N_TOKENS, 1), jnp.float32),
            jax.ShapeDtypeStruct((N_TOKENS, DH), jnp.int32),
        ],
    )(x, router_w, router_b.reshape(1, NUM_EXPERTS))

    e2d = jnp.concatenate(
        [i1.reshape(TROWS, 128), i2.reshape(TROWS, 128)], axis=0)
    w2d = jnp.concatenate(
        [w1.reshape(TROWS, 128), w2.reshape(TROWS, 128)], axis=0)

    pos2d, blk2d = pl.pallas_call(
        _dispatch_body,
        out_shape=[
            jax.ShapeDtypeStruct((AROWS, 128), jnp.int32),
            jax.ShapeDtypeStruct((1, 128), jnp.int32),
        ],
    )(e2d)
    blk_expert = blk2d.reshape(128)[:NBLK]

    xs, ws = _make_scatter_kernel()(xi, pos2d, w2d)

    y = pl.pallas_call(
        _gmm_body,
        grid_spec=pltpu.PrefetchScalarGridSpec(
            num_scalar_prefetch=1,
            grid=(NBLK,),
            in_specs=[
                pl.BlockSpec((B, DH), lambda b, be: (b, 0)),
                pl.BlockSpec((1, D_MODEL, D_MODEL), lambda b, be: (be[b], 0, 0)),
                pl.BlockSpec((1, 1, D_MODEL), lambda b, be: (be[b], 0, 0)),
                pl.BlockSpec((B, 1), lambda b, be: (b, 0)),
            ],
            out_specs=pl.BlockSpec((B, DH), lambda b, be: (b, 0)),
        ),
        out_shape=jax.ShapeDtypeStruct((M_PAD, DH), jnp.int32),
        compiler_params=pltpu.CompilerParams(
            dimension_semantics=("arbitrary",),
        ),
    )(blk_expert, xs, expert_w,
      expert_b.reshape(NUM_EXPERTS, 1, D_MODEL),
      ws.reshape(M_PAD, 1))

    y1, y2 = _make_combine_kernel()(y, pos2d)

    return pl.pallas_call(
        _add_body,
        grid=(4,),
        in_specs=[
            pl.BlockSpec((N_TOKENS // 4, DH), lambda t: (t, 0)),
            pl.BlockSpec((N_TOKENS // 4, DH), lambda t: (t, 0)),
        ],
        out_specs=pl.BlockSpec((N_TOKENS // 4, D_MODEL), lambda t: (t, 0)),
        out_shape=jax.ShapeDtypeStruct((N_TOKENS, D_MODEL), jnp.float32),
    )(y1, y2)


# speedup vs baseline: 5.0824x; 1.0749x over previous
import functools

import jax
import jax.numpy as jnp
from jax import lax
from jax.experimental import pallas as pl
from jax.experimental.pallas import tpu as pltpu
from jax.experimental.pallas import tpu_sc as plsc

N_TOKENS = 4096
D_MODEL = 1024
NUM_EXPERTS = 8
TB = 512
B = 256
M = 2 * N_TOKENS
NBLK = M // B + NUM_EXPERTS
M_PAD = NBLK * B
AROWS = M // 128
TROWS = N_TOKENS // 128
CH = 64
DH = D_MODEL // 2
HMASK = -65536


def _pack(lo_f32, hi_f32):
    lo_i = lax.bitcast_convert_type(
        lo_f32.astype(jnp.bfloat16).astype(jnp.float32), jnp.int32)
    hi_i = lax.bitcast_convert_type(
        hi_f32.astype(jnp.bfloat16).astype(jnp.float32), jnp.int32)
    return lax.shift_right_logical(lo_i, 16) | (hi_i & HMASK)


def _unpack(pk):
    lo = lax.bitcast_convert_type(lax.shift_left(pk, 16), jnp.float32)
    hi = lax.bitcast_convert_type(pk & HMASK, jnp.float32)
    return lo, hi


def _router_body(x_ref, rw_ref, rb_ref, xi_ref, w2d_ref, pos_ref, blk_ref,
                 e_acc):
    logits = jax.lax.dot_general(
        x_ref[...].astype(jnp.bfloat16), rw_ref[...].astype(jnp.bfloat16),
        (((1,), (1,)), ((), ())),
        preferred_element_type=jnp.float32,
    ) + rb_ref[...]
    m = jnp.max(logits, axis=-1, keepdims=True)
    eu = jnp.exp(logits - m)
    p = eu / jnp.sum(eu, axis=-1, keepdims=True)
    iota = jax.lax.broadcasted_iota(jnp.int32, p.shape, 1)
    p1 = jnp.max(p, axis=-1, keepdims=True)
    i1 = jnp.argmax(p, axis=-1)[:, None].astype(jnp.int32)
    pm = jnp.where(iota == i1, -1.0, p)
    p2 = jnp.max(pm, axis=-1, keepdims=True)
    i2 = jnp.argmax(pm, axis=-1)[:, None].astype(jnp.int32)
    s = p1 + p2
    t = pl.program_id(0)
    rpb = TB // 128
    e_acc[pl.ds(t * rpb, rpb), :] = i1.reshape(rpb, 128)
    e_acc[pl.ds(TROWS + t * rpb, rpb), :] = i2.reshape(rpb, 128)
    w2d_ref[pl.ds(t * rpb, rpb), :] = (p1 / s).reshape(rpb, 128)
    w2d_ref[pl.ds(TROWS + t * rpb, rpb), :] = (p2 / s).reshape(rpb, 128)
    xi_ref[...] = _pack(x_ref[:, :DH], x_ref[:, DH:])

    @pl.when(t == (N_TOKENS // TB) - 1)
    def _dispatch():
        _dispatch_compute(e_acc[...], pos_ref, blk_ref)


def _dispatch_compute(e2d, pos_ref, blk_ref):
    pos = jnp.zeros((AROWS, 128), jnp.int32)
    blk_acc = jnp.zeros((1, 128), jnp.int32)
    lane = jax.lax.broadcasted_iota(jnp.int32, (1, 128), 1)
    start_rows = jnp.int32(0)
    start_blk = jnp.int32(0)
    for e in range(NUM_EXPERTS):
        msk = (e2d == e).astype(jnp.int32)
        c = msk
        for s in (1, 2, 4, 8, 16, 32, 64):
            c = c + jnp.concatenate(
                [jnp.zeros((AROWS, s), jnp.int32), c[:, :128 - s]], axis=1)
        tot = c[:, 127:128]
        t = tot
        for s in (1, 2, 4, 8, 16, 32):
            t = t + jnp.concatenate(
                [jnp.zeros((s, 1), jnp.int32), t[:AROWS - s, :]], axis=0)
        rank_incl = c + (t - tot)
        cnt = jnp.sum(msk)
        blk_acc = blk_acc + (lane >= start_blk).astype(jnp.int32)
        pos = pos + msk * (rank_incl - 1 + start_rows)
        nblk_e = (cnt + (B - 1)) // B
        start_blk = start_blk + nblk_e
        start_rows = start_rows + nblk_e * B
    pos_ref[...] = pos
    blk_ref[...] = jnp.minimum(blk_acc - 1, NUM_EXPERTS - 1)


def _make_scatter_kernel():
    mesh = plsc.VectorSubcoreMesh(core_axis_name="c", subcore_axis_name="s")

    @functools.partial(
        pl.kernel, mesh=mesh,
        out_type=[
            jax.ShapeDtypeStruct((M_PAD, DH), jnp.int32),
            jax.ShapeDtypeStruct((M_PAD,), jnp.float32),
        ],
        scratch_types=[
            pltpu.VMEM((CH,), jnp.int32),
            pltpu.VMEM((CH,), jnp.int32),
            pltpu.VMEM((CH,), jnp.float32),
            pltpu.VMEM((CH,), jnp.float32),
            pltpu.VMEM((CH, DH), jnp.int32),
            pltpu.VMEM((CH, DH), jnp.int32),
            pltpu.SemaphoreType.DMA,
            pltpu.SemaphoreType.DMA,
            pltpu.SemaphoreType.DMA,
            pltpu.SemaphoreType.DMA,
            pltpu.SemaphoreType.DMA,
            pltpu.SemaphoreType.DMA,
        ],
    )
    def scatter_kernel(x_hbm, pos_hbm, w_hbm, xs_hbm, ws_hbm,
                       idx_a, idx_b, w_a, w_b, rows_a, rows_b,
                       sl_a, sl_b, ss_a, ss_b, sw_a, sw_b):
        wid = lax.axis_index("s") * 2 + lax.axis_index("c")
        idx_v = [idx_a, idx_b]
        w_v = [w_a, w_b]
        rows_v = [rows_a, rows_b]
        sl = [sl_a, sl_b]
        ss = [ss_a, ss_b]
        sw = [sw_a, sw_b]
        scat = [None, None]
        wscat = [None, None]
        for q in range(4):
            r = wid * 2 + q // 2
            h = (q % 2) * CH
            tok0 = (r % TROWS) * 128 + h
            buf = q % 2
            if scat[buf] is not None:
                scat[buf].wait()
                wscat[buf].wait()
            pltpu.sync_copy(pos_hbm.at[r, pl.ds(h, CH)], idx_v[buf])
            pltpu.sync_copy(w_hbm.at[r, pl.ds(h, CH)], w_v[buf])
            pltpu.async_copy(
                x_hbm.at[pl.ds(tok0, CH)], rows_v[buf], sl[buf]).wait()
            scat[buf] = pltpu.async_copy(
                rows_v[buf], xs_hbm.at[idx_v[buf]], ss[buf])
            wscat[buf] = pltpu.async_copy(
                w_v[buf], ws_hbm.at[idx_v[buf]], sw[buf])
        for b in (0, 1):
            scat[b].wait()
            wscat[b].wait()

    return scatter_kernel


def _gmm_body(be_smem, xs_ref, w_ref, b_ref, ws_ref, y_ref):
    a, bb = _unpack(xs_ref[...])
    w = w_ref[0]
    pre = jax.lax.dot_general(
        a.astype(jnp.bfloat16), w[:, :DH].astype(jnp.bfloat16),
        (((1,), (1,)), ((), ())),
        preferred_element_type=jnp.float32,
    ) + jax.lax.dot_general(
        bb.astype(jnp.bfloat16), w[:, DH:].astype(jnp.bfloat16),
        (((1,), (1,)), ((), ())),
        preferred_element_type=jnp.float32,
    ) + b_ref[0, 0][None, :]
    act = 0.5 * pre * (1.0 + jax.lax.erf(pre * 0.7071067811865476))
    y = act * ws_ref[...]
    y_ref[...] = _pack(y[:, :DH], y[:, DH:])


def _make_combine_kernel():
    mesh = plsc.VectorSubcoreMesh(core_axis_name="c", subcore_axis_name="s")

    @functools.partial(
        pl.kernel, mesh=mesh,
        out_type=[
            jax.ShapeDtypeStruct((N_TOKENS, DH), jnp.int32),
            jax.ShapeDtypeStruct((N_TOKENS, DH), jnp.int32),
        ],
        scratch_types=[
            pltpu.VMEM((CH,), jnp.int32),
            pltpu.VMEM((CH,), jnp.int32),
            pltpu.VMEM((CH, DH), jnp.int32),
            pltpu.VMEM((CH, DH), jnp.int32),
            pltpu.SemaphoreType.DMA,
            pltpu.SemaphoreType.DMA,
            pltpu.SemaphoreType.DMA,
            pltpu.SemaphoreType.DMA,
        ],
    )
    def combine_kernel(y_hbm, pos_hbm, y1_hbm, y2_hbm,
                       idx_a, idx_b, rows_a, rows_b, sg_a, sg_b, sw_a, sw_b):
        wid = lax.axis_index("s") * 2 + lax.axis_index("c")
        idx_v = [idx_a, idx_b]
        rows_v = [rows_a, rows_b]
        sg = [sg_a, sg_b]
        sw = [sw_a, sw_b]
        outs = [y1_hbm, y2_hbm]
        wr = [None, None]
        for q in range(4):
            k = q // 2
            h = q % 2
            buf = q % 2
            base = wid * 128 + h * CH
            if wr[buf] is not None:
                wr[buf].wait()
            pltpu.sync_copy(
                pos_hbm.at[k * TROWS + wid, pl.ds(h * CH, CH)], idx_v[buf])
            pltpu.async_copy(
                y_hbm.at[idx_v[buf]], rows_v[buf], sg[buf]).wait()
            wr[buf] = pltpu.async_copy(
                rows_v[buf], outs[k].at[pl.ds(base, CH)], sw[buf])
        for b in (0, 1):
            wr[b].wait()

    return combine_kernel


def _add_body(y1_ref, y2_ref, out_ref):
    lo1, hi1 = _unpack(y1_ref[...])
    lo2, hi2 = _unpack(y2_ref[...])
    out_ref[:, :DH] = lo1 + lo2
    out_ref[:, DH:] = hi1 + hi2


@jax.jit
def kernel(x, router_w, router_b, expert_w, expert_b):
    xi, w2d, pos2d, blk2d = pl.pallas_call(
        _router_body,
        grid=(N_TOKENS // TB,),
        in_specs=[
            pl.BlockSpec((TB, D_MODEL), lambda t: (t, 0)),
            pl.BlockSpec((NUM_EXPERTS, D_MODEL), lambda t: (0, 0)),
            pl.BlockSpec((1, NUM_EXPERTS), lambda t: (0, 0)),
        ],
        out_specs=[
            pl.BlockSpec((TB, DH), lambda t: (t, 0)),
            pl.BlockSpec((AROWS, 128), lambda t: (0, 0)),
            pl.BlockSpec((AROWS, 128), lambda t: (0, 0)),
            pl.BlockSpec((1, 128), lambda t: (0, 0)),
        ],
        out_shape=[
            jax.ShapeDtypeStruct((N_TOKENS, DH), jnp.int32),
            jax.ShapeDtypeStruct((AROWS, 128), jnp.float32),
            jax.ShapeDtypeStruct((AROWS, 128), jnp.int32),
            jax.ShapeDtypeStruct((1, 128), jnp.int32),
        ],
        scratch_shapes=[pltpu.VMEM((AROWS, 128), jnp.int32)],
    )(x, router_w, router_b.reshape(1, NUM_EXPERTS))
    blk_expert = blk2d.reshape(128)[:NBLK]

    xs, ws = _make_scatter_kernel()(xi, pos2d, w2d)

    y = pl.pallas_call(
        _gmm_body,
        grid_spec=pltpu.PrefetchScalarGridSpec(
            num_scalar_prefetch=1,
            grid=(NBLK,),
            in_specs=[
                pl.BlockSpec((B, DH), lambda b, be: (b, 0)),
                pl.BlockSpec((1, D_MODEL, D_MODEL), lambda b, be: (be[b], 0, 0)),
                pl.BlockSpec((1, 1, D_MODEL), lambda b, be: (be[b], 0, 0)),
                pl.BlockSpec((B, 1), lambda b, be: (b, 0)),
            ],
            out_specs=pl.BlockSpec((B, DH), lambda b, be: (b, 0)),
        ),
        out_shape=jax.ShapeDtypeStruct((M_PAD, DH), jnp.int32),
        compiler_params=pltpu.CompilerParams(
            dimension_semantics=("arbitrary",),
        ),
    )(blk_expert, xs, expert_w,
      expert_b.reshape(NUM_EXPERTS, 1, D_MODEL),
      ws.reshape(M_PAD, 1))

    y1, y2 = _make_combine_kernel()(y, pos2d)

    return pl.pallas_call(
        _add_body,
        grid=(4,),
        in_specs=[
            pl.BlockSpec((N_TOKENS // 4, DH), lambda t: (t, 0)),
            pl.BlockSpec((N_TOKENS // 4, DH), lambda t: (t, 0)),
        ],
        out_specs=pl.BlockSpec((N_TOKENS // 4, D_MODEL), lambda t: (t, 0)),
        out_shape=jax.ShapeDtypeStruct((N_TOKENS, D_MODEL), jnp.float32),
    )(y1, y2)


# speedup vs baseline: 6.1164x; 1.2034x over previous
import functools

import jax
import jax.numpy as jnp
from jax import lax
from jax.experimental import pallas as pl
from jax.experimental.pallas import tpu as pltpu
from jax.experimental.pallas import tpu_sc as plsc

N_TOKENS = 4096
D_MODEL = 1024
NUM_EXPERTS = 8
TB = 512
B = 256
M = 2 * N_TOKENS
NBLK = M // B + NUM_EXPERTS
M_PAD = NBLK * B
AROWS = M // 128
TROWS = N_TOKENS // 128
CH = 64
DH = D_MODEL // 2
HMASK = -65536


def _pack(lo_f32, hi_f32):
    lo_i = lax.bitcast_convert_type(
        lo_f32.astype(jnp.bfloat16).astype(jnp.float32), jnp.int32)
    hi_i = lax.bitcast_convert_type(
        hi_f32.astype(jnp.bfloat16).astype(jnp.float32), jnp.int32)
    return lax.shift_right_logical(lo_i, 16) | (hi_i & HMASK)


def _unpack(pk):
    lo = lax.bitcast_convert_type(lax.shift_left(pk, 16), jnp.float32)
    hi = lax.bitcast_convert_type(pk & HMASK, jnp.float32)
    return lo, hi


def _router_body(x_ref, rw_ref, rb_ref, xi_ref, w1_ref, w2_ref, pos_ref,
                 blk_ref, e_acc):
    logits = jax.lax.dot_general(
        x_ref[...].astype(jnp.bfloat16), rw_ref[...].astype(jnp.bfloat16),
        (((1,), (1,)), ((), ())),
        preferred_element_type=jnp.float32,
    ) + rb_ref[...]
    m = jnp.max(logits, axis=-1, keepdims=True)
    eu = jnp.exp(logits - m)
    p = eu / jnp.sum(eu, axis=-1, keepdims=True)
    iota = jax.lax.broadcasted_iota(jnp.int32, p.shape, 1)
    p1 = jnp.max(p, axis=-1, keepdims=True)
    i1 = jnp.argmax(p, axis=-1)[:, None].astype(jnp.int32)
    pm = jnp.where(iota == i1, -1.0, p)
    p2 = jnp.max(pm, axis=-1, keepdims=True)
    i2 = jnp.argmax(pm, axis=-1)[:, None].astype(jnp.int32)
    s = p1 + p2
    t = pl.program_id(0)
    rpb = TB // 128
    e_acc[pl.ds(t * rpb, rpb), :] = i1.reshape(rpb, 128)
    e_acc[pl.ds(TROWS + t * rpb, rpb), :] = i2.reshape(rpb, 128)
    w1_ref[...] = p1 / s
    w2_ref[...] = p2 / s
    xi_ref[...] = _pack(x_ref[:, :DH], x_ref[:, DH:])

    @pl.when(t == (N_TOKENS // TB) - 1)
    def _dispatch():
        _dispatch_compute(e_acc[...], pos_ref, blk_ref)


def _dispatch_compute(e2d, pos_ref, blk_ref):
    pos = jnp.zeros((AROWS, 128), jnp.int32)
    blk_acc = jnp.zeros((1, 128), jnp.int32)
    lane = jax.lax.broadcasted_iota(jnp.int32, (1, 128), 1)
    start_rows = jnp.int32(0)
    start_blk = jnp.int32(0)
    for e in range(NUM_EXPERTS):
        msk = (e2d == e).astype(jnp.int32)
        c = msk
        for s in (1, 2, 4, 8, 16, 32, 64):
            c = c + jnp.concatenate(
                [jnp.zeros((AROWS, s), jnp.int32), c[:, :128 - s]], axis=1)
        tot = c[:, 127:128]
        t = tot
        for s in (1, 2, 4, 8, 16, 32):
            t = t + jnp.concatenate(
                [jnp.zeros((s, 1), jnp.int32), t[:AROWS - s, :]], axis=0)
        rank_incl = c + (t - tot)
        cnt = jnp.sum(msk)
        blk_acc = blk_acc + (lane >= start_blk).astype(jnp.int32)
        pos = pos + msk * (rank_incl - 1 + start_rows)
        nblk_e = (cnt + (B - 1)) // B
        start_blk = start_blk + nblk_e
        start_rows = start_rows + nblk_e * B
    pos_ref[...] = pos
    blk_ref[...] = jnp.minimum(blk_acc - 1, NUM_EXPERTS - 1)


def _make_scatter_kernel():
    mesh = plsc.VectorSubcoreMesh(core_axis_name="c", subcore_axis_name="s")

    @functools.partial(
        pl.kernel, mesh=mesh,
        out_type=jax.ShapeDtypeStruct((M_PAD, DH), jnp.int32),
        scratch_types=[
            pltpu.VMEM((4, CH), jnp.int32),
            pltpu.VMEM((CH, DH), jnp.int32),
            pltpu.VMEM((CH, DH), jnp.int32),
            pltpu.SemaphoreType.DMA,
            pltpu.SemaphoreType.DMA,
            pltpu.SemaphoreType.DMA,
            pltpu.SemaphoreType.DMA,
        ],
    )
    def scatter_kernel(x_hbm, pos_hbm, xs_hbm,
                       idx_v, rows_a, rows_b, sl_a, sl_b, ss_a, ss_b):
        wid = lax.axis_index("s") * 2 + lax.axis_index("c")
        rows_v = [rows_a, rows_b]
        sl = [sl_a, sl_b]
        ss = [ss_a, ss_b]
        scat = [None, None]
        pltpu.sync_copy(pos_hbm.at[pl.ds(wid * 4, 4)], idx_v)
        for q in range(4):
            r = wid * 2 + q // 2
            h = (q % 2) * CH
            tok0 = (r % TROWS) * 128 + h
            buf = q % 2
            if scat[buf] is not None:
                scat[buf].wait()
            pltpu.async_copy(
                x_hbm.at[pl.ds(tok0, CH)], rows_v[buf], sl[buf]).wait()
            scat[buf] = pltpu.async_copy(
                rows_v[buf], xs_hbm.at[idx_v.at[q]], ss[buf])
        for b in (0, 1):
            scat[b].wait()

    return scatter_kernel


def _gmm_body(be_smem, xs_ref, w_ref, b_ref, y_ref):
    a, bb = _unpack(xs_ref[...])
    w = w_ref[0]
    pre = jax.lax.dot_general(
        a.astype(jnp.bfloat16), w[:, :DH].astype(jnp.bfloat16),
        (((1,), (1,)), ((), ())),
        preferred_element_type=jnp.float32,
    ) + jax.lax.dot_general(
        bb.astype(jnp.bfloat16), w[:, DH:].astype(jnp.bfloat16),
        (((1,), (1,)), ((), ())),
        preferred_element_type=jnp.float32,
    ) + b_ref[0, 0][None, :]
    act = 0.5 * pre * (1.0 + jax.lax.erf(pre * 0.7071067811865476))
    y_ref[...] = _pack(act[:, :DH], act[:, DH:])


def _make_combine_kernel():
    mesh = plsc.VectorSubcoreMesh(core_axis_name="c", subcore_axis_name="s")

    @functools.partial(
        pl.kernel, mesh=mesh,
        out_type=[
            jax.ShapeDtypeStruct((N_TOKENS, DH), jnp.int32),
            jax.ShapeDtypeStruct((N_TOKENS, DH), jnp.int32),
        ],
        scratch_types=[
            pltpu.VMEM((CH,), jnp.int32),
            pltpu.VMEM((CH,), jnp.int32),
            pltpu.VMEM((CH, DH), jnp.int32),
            pltpu.VMEM((CH, DH), jnp.int32),
            pltpu.SemaphoreType.DMA,
            pltpu.SemaphoreType.DMA,
            pltpu.SemaphoreType.DMA,
            pltpu.SemaphoreType.DMA,
        ],
    )
    def combine_kernel(y_hbm, pos_hbm, y1_hbm, y2_hbm,
                       idx_a, idx_b, rows_a, rows_b, sg_a, sg_b, sw_a, sw_b):
        wid = lax.axis_index("s") * 2 + lax.axis_index("c")
        idx_v = [idx_a, idx_b]
        rows_v = [rows_a, rows_b]
        sg = [sg_a, sg_b]
        sw = [sw_a, sw_b]
        outs = [y1_hbm, y2_hbm]
        wr = [None, None]
        for q in range(4):
            k = q // 2
            h = q % 2
            buf = q % 2
            base = wid * 128 + h * CH
            if wr[buf] is not None:
                wr[buf].wait()
            pltpu.sync_copy(
                pos_hbm.at[k * TROWS + wid, pl.ds(h * CH, CH)], idx_v[buf])
            pltpu.async_copy(
                y_hbm.at[idx_v[buf]], rows_v[buf], sg[buf]).wait()
            wr[buf] = pltpu.async_copy(
                rows_v[buf], outs[k].at[pl.ds(base, CH)], sw[buf])
        for b in (0, 1):
            wr[b].wait()

    return combine_kernel


def _add_body(y1_ref, y2_ref, w1_ref, w2_ref, out_ref):
    lo1, hi1 = _unpack(y1_ref[...])
    lo2, hi2 = _unpack(y2_ref[...])
    w1 = w1_ref[...]
    w2 = w2_ref[...]
    out_ref[:, :DH] = w1 * lo1 + w2 * lo2
    out_ref[:, DH:] = w1 * hi1 + w2 * hi2


@jax.jit
def kernel(x, router_w, router_b, expert_w, expert_b):
    xi, w1, w2, pos2d, blk2d = pl.pallas_call(
        _router_body,
        grid=(N_TOKENS // TB,),
        in_specs=[
            pl.BlockSpec((TB, D_MODEL), lambda t: (t, 0)),
            pl.BlockSpec((NUM_EXPERTS, D_MODEL), lambda t: (0, 0)),
            pl.BlockSpec((1, NUM_EXPERTS), lambda t: (0, 0)),
        ],
        out_specs=[
            pl.BlockSpec((TB, DH), lambda t: (t, 0)),
            pl.BlockSpec((TB, 1), lambda t: (t, 0)),
            pl.BlockSpec((TB, 1), lambda t: (t, 0)),
            pl.BlockSpec((AROWS, 128), lambda t: (0, 0)),
            pl.BlockSpec((1, 128), lambda t: (0, 0)),
        ],
        out_shape=[
            jax.ShapeDtypeStruct((N_TOKENS, DH), jnp.int32),
            jax.ShapeDtypeStruct((N_TOKENS, 1), jnp.float32),
            jax.ShapeDtypeStruct((N_TOKENS, 1), jnp.float32),
            jax.ShapeDtypeStruct((AROWS, 128), jnp.int32),
            jax.ShapeDtypeStruct((1, 128), jnp.int32),
        ],
        scratch_shapes=[pltpu.VMEM((AROWS, 128), jnp.int32)],
    )(x, router_w, router_b.reshape(1, NUM_EXPERTS))
    blk_expert = blk2d.reshape(128)[:NBLK]

    xs = _make_scatter_kernel()(xi, pos2d.reshape(2 * AROWS, CH))

    y = pl.pallas_call(
        _gmm_body,
        grid_spec=pltpu.PrefetchScalarGridSpec(
            num_scalar_prefetch=1,
            grid=(NBLK,),
            in_specs=[
                pl.BlockSpec((B, DH), lambda b, be: (b, 0)),
                pl.BlockSpec((1, D_MODEL, D_MODEL), lambda b, be: (be[b], 0, 0)),
                pl.BlockSpec((1, 1, D_MODEL), lambda b, be: (be[b], 0, 0)),
            ],
            out_specs=pl.BlockSpec((B, DH), lambda b, be: (b, 0)),
        ),
        out_shape=jax.ShapeDtypeStruct((M_PAD, DH), jnp.int32),
        compiler_params=pltpu.CompilerParams(
            dimension_semantics=("arbitrary",),
        ),
    )(blk_expert, xs, expert_w,
      expert_b.reshape(NUM_EXPERTS, 1, D_MODEL))

    y1, y2 = _make_combine_kernel()(y, pos2d)

    return pl.pallas_call(
        _add_body,
        grid=(4,),
        in_specs=[
            pl.BlockSpec((N_TOKENS // 4, DH), lambda t: (t, 0)),
            pl.BlockSpec((N_TOKENS // 4, DH), lambda t: (t, 0)),
            pl.BlockSpec((N_TOKENS // 4, 1), lambda t: (t, 0)),
            pl.BlockSpec((N_TOKENS // 4, 1), lambda t: (t, 0)),
        ],
        out_specs=pl.BlockSpec((N_TOKENS // 4, D_MODEL), lambda t: (t, 0)),
        out_shape=jax.ShapeDtypeStruct((N_TOKENS, D_MODEL), jnp.float32),
    )(y1, y2, w1, w2)


# speedup vs baseline: 6.1356x; 1.0031x over previous
import functools

import jax
import jax.numpy as jnp
from jax import lax
from jax.experimental import pallas as pl
from jax.experimental.pallas import tpu as pltpu
from jax.experimental.pallas import tpu_sc as plsc

N_TOKENS = 4096
D_MODEL = 1024
NUM_EXPERTS = 8
TB = 512
B = 256
M = 2 * N_TOKENS
NBLK = M // B + NUM_EXPERTS
M_PAD = NBLK * B
AROWS = M // 128
TROWS = N_TOKENS // 128
CH = 64
DH = D_MODEL // 2
HMASK = -65536


def _pack(lo_f32, hi_f32):
    lo_i = lax.bitcast_convert_type(
        lo_f32.astype(jnp.bfloat16).astype(jnp.float32), jnp.int32)
    hi_i = lax.bitcast_convert_type(
        hi_f32.astype(jnp.bfloat16).astype(jnp.float32), jnp.int32)
    return lax.shift_right_logical(lo_i, 16) | (hi_i & HMASK)


def _unpack(pk):
    lo = lax.bitcast_convert_type(lax.shift_left(pk, 16), jnp.float32)
    hi = lax.bitcast_convert_type(pk & HMASK, jnp.float32)
    return lo, hi


def _router_body(x_ref, rw_ref, rb_ref, xi_ref, w1_ref, w2_ref, pos_ref,
                 blk_ref, e_acc):
    logits = jax.lax.dot_general(
        x_ref[...].astype(jnp.bfloat16), rw_ref[...].astype(jnp.bfloat16),
        (((1,), (1,)), ((), ())),
        preferred_element_type=jnp.float32,
    ) + rb_ref[...]
    m = jnp.max(logits, axis=-1, keepdims=True)
    eu = jnp.exp(logits - m)
    p = eu / jnp.sum(eu, axis=-1, keepdims=True)
    iota = jax.lax.broadcasted_iota(jnp.int32, p.shape, 1)
    p1 = jnp.max(p, axis=-1, keepdims=True)
    i1 = jnp.argmax(p, axis=-1)[:, None].astype(jnp.int32)
    pm = jnp.where(iota == i1, -1.0, p)
    p2 = jnp.max(pm, axis=-1, keepdims=True)
    i2 = jnp.argmax(pm, axis=-1)[:, None].astype(jnp.int32)
    s = p1 + p2
    t = pl.program_id(0)
    rpb = TB // 128
    e_acc[pl.ds(t * rpb, rpb), :] = i1.reshape(rpb, 128)
    e_acc[pl.ds(TROWS + t * rpb, rpb), :] = i2.reshape(rpb, 128)
    w1_ref[...] = p1 / s
    w2_ref[...] = p2 / s
    xi_ref[...] = _pack(x_ref[:, :DH], x_ref[:, DH:])

    @pl.when(t == (N_TOKENS // TB) - 1)
    def _dispatch():
        _dispatch_compute(e_acc[...], pos_ref, blk_ref)


def _dispatch_compute(e2d, pos_ref, blk_ref):
    pos = jnp.zeros((AROWS, 128), jnp.int32)
    blk_acc = jnp.zeros((1, 128), jnp.int32)
    lane = jax.lax.broadcasted_iota(jnp.int32, (1, 128), 1)
    start_rows = jnp.int32(0)
    start_blk = jnp.int32(0)
    for e in range(NUM_EXPERTS):
        msk = (e2d == e).astype(jnp.int32)
        c = msk
        for s in (1, 2, 4, 8, 16, 32, 64):
            c = c + jnp.concatenate(
                [jnp.zeros((AROWS, s), jnp.int32), c[:, :128 - s]], axis=1)
        tot = c[:, 127:128]
        t = tot
        for s in (1, 2, 4, 8, 16, 32):
            t = t + jnp.concatenate(
                [jnp.zeros((s, 1), jnp.int32), t[:AROWS - s, :]], axis=0)
        rank_incl = c + (t - tot)
        cnt = jnp.sum(msk)
        blk_acc = blk_acc + (lane >= start_blk).astype(jnp.int32)
        pos = pos + msk * (rank_incl - 1 + start_rows)
        nblk_e = (cnt + (B - 1)) // B
        start_blk = start_blk + nblk_e
        start_rows = start_rows + nblk_e * B
    pos_ref[...] = pos
    blk_ref[...] = jnp.minimum(blk_acc - 1, NUM_EXPERTS - 1)


def _make_scatter_kernel():
    mesh = plsc.VectorSubcoreMesh(core_axis_name="c", subcore_axis_name="s")

    @functools.partial(
        pl.kernel, mesh=mesh,
        out_type=jax.ShapeDtypeStruct((M_PAD, DH), jnp.int32),
        scratch_types=[
            pltpu.VMEM((4, CH), jnp.int32),
            pltpu.VMEM((CH, DH), jnp.int32),
            pltpu.VMEM((CH, DH), jnp.int32),
            pltpu.SemaphoreType.DMA,
            pltpu.SemaphoreType.DMA,
            pltpu.SemaphoreType.DMA,
            pltpu.SemaphoreType.DMA,
        ],
    )
    def scatter_kernel(x_hbm, pos_hbm, xs_hbm,
                       idx_v, rows_a, rows_b, sl_a, sl_b, ss_a, ss_b):
        wid = lax.axis_index("s") * 2 + lax.axis_index("c")
        rows_v = [rows_a, rows_b]
        sl = [sl_a, sl_b]
        ss = [ss_a, ss_b]
        scat = [None, None]
        pltpu.sync_copy(pos_hbm.at[pl.ds(wid * 4, 4)], idx_v)
        for q in range(4):
            r = wid * 2 + q // 2
            h = (q % 2) * CH
            tok0 = (r % TROWS) * 128 + h
            buf = q % 2
            if scat[buf] is not None:
                scat[buf].wait()
            pltpu.async_copy(
                x_hbm.at[pl.ds(tok0, CH)], rows_v[buf], sl[buf]).wait()
            scat[buf] = pltpu.async_copy(
                rows_v[buf], xs_hbm.at[idx_v.at[q]], ss[buf])
        for b in (0, 1):
            scat[b].wait()

    return scatter_kernel


def _gmm_body(be_smem, xs_ref, w_ref, b_ref, y_ref):
    a, bb = _unpack(xs_ref[...])
    w = w_ref[0]
    pre = jax.lax.dot_general(
        a.astype(jnp.bfloat16), w[:, :DH].astype(jnp.bfloat16),
        (((1,), (1,)), ((), ())),
        preferred_element_type=jnp.float32,
    ) + jax.lax.dot_general(
        bb.astype(jnp.bfloat16), w[:, DH:].astype(jnp.bfloat16),
        (((1,), (1,)), ((), ())),
        preferred_element_type=jnp.float32,
    ) + b_ref[0, 0][None, :]
    act = 0.5 * pre * (1.0 + jax.lax.erf(pre * 0.7071067811865476))
    y_ref[...] = _pack(act[:, :DH], act[:, DH:])


def _make_combine_kernel():
    mesh = plsc.VectorSubcoreMesh(core_axis_name="c", subcore_axis_name="s")

    @functools.partial(
        pl.kernel, mesh=mesh,
        out_type=[
            jax.ShapeDtypeStruct((N_TOKENS, DH), jnp.int32),
            jax.ShapeDtypeStruct((N_TOKENS, DH), jnp.int32),
        ],
        scratch_types=[
            pltpu.VMEM((4, CH), jnp.int32),
            pltpu.VMEM((CH, DH), jnp.int32),
            pltpu.VMEM((CH, DH), jnp.int32),
            pltpu.SemaphoreType.DMA,
            pltpu.SemaphoreType.DMA,
            pltpu.SemaphoreType.DMA,
            pltpu.SemaphoreType.DMA,
        ],
    )
    def combine_kernel(y_hbm, pos_hbm, y1_hbm, y2_hbm,
                       idx_v, rows_a, rows_b, sg_a, sg_b, sw_a, sw_b):
        wid = lax.axis_index("s") * 2 + lax.axis_index("c")
        rows_v = [rows_a, rows_b]
        sg = [sg_a, sg_b]
        sw = [sw_a, sw_b]
        outs = [y1_hbm, y2_hbm]
        wr = [None, None]
        pltpu.sync_copy(pos_hbm.at[pl.ds(2 * wid, 2)], idx_v.at[pl.ds(0, 2)])
        pltpu.sync_copy(
            pos_hbm.at[pl.ds(2 * TROWS + 2 * wid, 2)], idx_v.at[pl.ds(2, 2)])
        for q in range(4):
            k = q // 2
            h = q % 2
            buf = q % 2
            base = wid * 128 + h * CH
            if wr[buf] is not None:
                wr[buf].wait()
            pltpu.async_copy(
                y_hbm.at[idx_v.at[q]], rows_v[buf], sg[buf]).wait()
            wr[buf] = pltpu.async_copy(
                rows_v[buf], outs[k].at[pl.ds(base, CH)], sw[buf])
        for b in (0, 1):
            wr[b].wait()

    return combine_kernel


def _add_body(y1_ref, y2_ref, w1_ref, w2_ref, out_ref):
    lo1, hi1 = _unpack(y1_ref[...])
    lo2, hi2 = _unpack(y2_ref[...])
    w1 = w1_ref[...]
    w2 = w2_ref[...]
    out_ref[:, :DH] = w1 * lo1 + w2 * lo2
    out_ref[:, DH:] = w1 * hi1 + w2 * hi2


@jax.jit
def kernel(x, router_w, router_b, expert_w, expert_b):
    xi, w1, w2, pos2d, blk2d = pl.pallas_call(
        _router_body,
        grid=(N_TOKENS // TB,),
        in_specs=[
            pl.BlockSpec((TB, D_MODEL), lambda t: (t, 0)),
            pl.BlockSpec((NUM_EXPERTS, D_MODEL), lambda t: (0, 0)),
            pl.BlockSpec((1, NUM_EXPERTS), lambda t: (0, 0)),
        ],
        out_specs=[
            pl.BlockSpec((TB, DH), lambda t: (t, 0)),
            pl.BlockSpec((TB, 1), lambda t: (t, 0)),
            pl.BlockSpec((TB, 1), lambda t: (t, 0)),
            pl.BlockSpec((AROWS, 128), lambda t: (0, 0)),
            pl.BlockSpec((1, 128), lambda t: (0, 0)),
        ],
        out_shape=[
            jax.ShapeDtypeStruct((N_TOKENS, DH), jnp.int32),
            jax.ShapeDtypeStruct((N_TOKENS, 1), jnp.float32),
            jax.ShapeDtypeStruct((N_TOKENS, 1), jnp.float32),
            jax.ShapeDtypeStruct((AROWS, 128), jnp.int32),
            jax.ShapeDtypeStruct((1, 128), jnp.int32),
        ],
        scratch_shapes=[pltpu.VMEM((AROWS, 128), jnp.int32)],
    )(x, router_w, router_b.reshape(1, NUM_EXPERTS))
    blk_expert = blk2d.reshape(128)[:NBLK]

    xs = _make_scatter_kernel()(xi, pos2d.reshape(2 * AROWS, CH))

    y = pl.pallas_call(
        _gmm_body,
        grid_spec=pltpu.PrefetchScalarGridSpec(
            num_scalar_prefetch=1,
            grid=(NBLK,),
            in_specs=[
                pl.BlockSpec((B, DH), lambda b, be: (b, 0)),
                pl.BlockSpec((1, D_MODEL, D_MODEL), lambda b, be: (be[b], 0, 0)),
                pl.BlockSpec((1, 1, D_MODEL), lambda b, be: (be[b], 0, 0)),
            ],
            out_specs=pl.BlockSpec((B, DH), lambda b, be: (b, 0)),
        ),
        out_shape=jax.ShapeDtypeStruct((M_PAD, DH), jnp.int32),
        compiler_params=pltpu.CompilerParams(
            dimension_semantics=("arbitrary",),
        ),
    )(blk_expert, xs, expert_w,
      expert_b.reshape(NUM_EXPERTS, 1, D_MODEL))

    y1, y2 = _make_combine_kernel()(y, pos2d.reshape(2 * AROWS, CH))

    return pl.pallas_call(
        _add_body,
        grid=(4,),
        in_specs=[
            pl.BlockSpec((N_TOKENS // 4, DH), lambda t: (t, 0)),
            pl.BlockSpec((N_TOKENS // 4, DH), lambda t: (t, 0)),
            pl.BlockSpec((N_TOKENS // 4, 1), lambda t: (t, 0)),
            pl.BlockSpec((N_TOKENS // 4, 1), lambda t: (t, 0)),
        ],
        out_specs=pl.BlockSpec((N_TOKENS // 4, D_MODEL), lambda t: (t, 0)),
        out_shape=jax.ShapeDtypeStruct((N_TOKENS, D_MODEL), jnp.float32),
    )(y1, y2, w1, w2)
